# baseline (speedup 1.0000x reference)
import functools

import jax
import jax.numpy as jnp
from jax import lax
from jax.experimental import pallas as pl
from jax.experimental.pallas import tpu as pltpu
from jax.experimental.pallas import tpu_sc as plsc

_NC = 2
_NS = 16
_L = 16
_F = 66
_D = 4
_RW = 8
_H1 = 16
_H2 = 8
_SPB = 64
_G = _SPB // _L

_OW1 = 0
_OB1 = _OW1 + _F * _D * _H1
_OW2 = _OB1 + _H1
_OB2 = _OW2 + _H2 * _H1
_OW3 = _OB2 + _H2
_OB3 = _OW3 + _H2
_WLEN = 4392


@functools.lru_cache(maxsize=None)
def _make_fused(B, n_idx):
    nw = _NC * _NS
    spw = B // nw
    n_chunks = spw // _SPB
    ch = _SPB * _F
    assert spw % _SPB == 0 and ch % 8 == 0

    mesh = plsc.VectorSubcoreMesh(
        core_axis_name="c", subcore_axis_name="s",
        num_cores=_NC, num_subcores=_NS)

    def compute_group(rv, w_v, g4):
        iota = lax.iota(jnp.int32, _L)
        rowbase = (iota + g4 * _L) * _F
        dcol = [jnp.full((_L,), d, jnp.int32) for d in range(_D + 1)]
        zero = jnp.zeros((_L,), jnp.float32)

        h1 = []
        bacc_out = None
        for half in range(2):
            def f_body(f, carry):
                accs = list(carry[:8])
                bacc = carry[8]
                idx0 = rowbase + f
                for d in range(_D):
                    xv = plsc.load_gather(rv, [idx0, dcol[d]])
                    wv = w_v[pl.ds((f * _D + d) * _H1 + half * 8, _L)]
                    for j8 in range(8):
                        accs[j8] = accs[j8] + xv * wv[j8]
                if half == 0:
                    bacc = bacc + plsc.load_gather(rv, [idx0, dcol[_D]])
                return tuple(accs) + (bacc,)

            out = lax.fori_loop(
                0, _F, f_body, (zero,) * 8 + (zero,), unroll=1)
            h1 += list(out[:8])
            if half == 0:
                bacc_out = out[8]

        b1v = w_v[pl.ds(_OB1, _L)]
        h1 = [jnp.maximum(h1[j] + b1v[j], 0.0) for j in range(_H1)]
        b2v = w_v[pl.ds(_OB2, _L)]
        h2 = []
        for k in range(_H2):
            wv2 = w_v[pl.ds(_OW2 + k * _H1, _L)]
            a = zero
            for j in range(_H1):
                a = a + h1[j] * wv2[j]
            h2.append(jnp.maximum(a + b2v[k], 0.0))
        w3v = w_v[pl.ds(_OW3, _L)]
        o = zero
        for k in range(_H2):
            o = o + h2[k] * w3v[k]
        return o + w3v[_OB3 - _OW3] + bacc_out

    @functools.partial(
        pl.kernel,
        out_type=jax.ShapeDtypeStruct((B,), jnp.float32),
        mesh=mesh,
        scratch_types=[
            pltpu.VMEM((ch,), jnp.int32),
            pltpu.VMEM((ch,), jnp.int32),
            pltpu.VMEM((ch, _RW), jnp.float32),
            pltpu.VMEM((ch, _RW), jnp.float32),
            pltpu.VMEM((_WLEN,), jnp.float32),
            pltpu.VMEM((spw,), jnp.float32),
            pltpu.SemaphoreType.DMA((2,)),
        ],
        compiler_params=pltpu.CompilerParams(
            use_tc_tiling_on_sc=False, needs_layout_passes=False),
    )
    def fused_k(tab_hbm, idx_hbm, wpack_hbm, out_hbm,
                i0_v, i1_v, r0_v, r1_v, w_v, out_v, gsem):
        wid = lax.axis_index("s") * _NC + lax.axis_index("c")
        sbase = wid * spw
        ibase = wid * spw * _F
        idx_bufs = (i0_v, i1_v)
        row_bufs = (r0_v, r1_v)

        pltpu.sync_copy(wpack_hbm, w_v)

        def start_gather(c):
            b = c % 2
            pltpu.sync_copy(
                idx_hbm.at[pl.ds(ibase + c * ch, ch)], idx_bufs[b])
            return pltpu.async_copy(
                tab_hbm.at[idx_bufs[b]], row_bufs[b], gsem.at[b])

        gathers = {0: start_gather(0)}
        for c in range(n_chunks):
            if c + 1 < n_chunks:
                gathers[c + 1] = start_gather(c + 1)
            gathers[c].wait()
            rv = row_bufs[c % 2]

            def g_body(g4, _, rv=rv, c=c):
                o = compute_group(rv, w_v, g4)
                out_v[pl.ds(c * _SPB + g4 * _L, _L)] = o
                return 0

            lax.fori_loop(0, _G, g_body, 0, unroll=1)

        pltpu.sync_copy(out_v, out_hbm.at[pl.ds(sbase, spw)])

    return fused_k


def kernel(fids_batch, emb_w, emb_b, W1, b1, W2, b2, W3, b3):
    B, F = fids_batch.shape
    V, D = emb_w.shape
    N = B * F

    tab = jnp.concatenate(
        [emb_w, emb_b[:, None], jnp.zeros((V, _RW - D - 1), jnp.float32)],
        axis=1)
    fids_flat = fids_batch.reshape(N)

    wpack = jnp.concatenate([
        W1.T.reshape(F * D * _H1),
        b1,
        W2.reshape(_H2 * _H1),
        b2,
        W3.reshape(_H2),
        b3,
        jnp.zeros((_WLEN - _OB3 - 1,), jnp.float32),
    ])

    return _make_fused(B, N)(tab, fids_flat, wpack)
